# baseline (speedup 1.0000x reference)
import functools

import jax
import jax.numpy as jnp
import numpy as np
from jax import lax
from jax.experimental import pallas as pl
from jax.experimental.pallas import tpu as pltpu
from jax.experimental.pallas import tpu_sc as plsc

N = 10000
D = 128
E = 320000
NC = 2
NS = 16
NW = NC * NS
CHUNK = 96
CHUNKS = 112
HALFN = 56
EP = NW * CHUNKS * CHUNK
NPAD = 10240
RPT = NPAD // NS
TCB = 2000

_PERM = np.array([c // 2 if c % 2 == 0 else 64 + c // 2 for c in range(D)],
                 np.int32)

_MESH = plsc.VectorSubcoreMesh(
    core_axis_name="c", subcore_axis_name="s", num_cores=NC, num_subcores=NS)


def _zero_rows(ref, nrows):
    zeros16 = jnp.zeros((16,), jnp.float32)

    def body(i, carry):
        for j in range(8):
            ref[i, pl.ds(j * 16, 16)] = zeros16
        return carry

    lax.fori_loop(0, nrows, body, None)


def _sc_deg_body(dst_hbm, out0, out1, dst_v, ones_v, buf_v, hist,
                 sem0, sem1, sem2, sem3):
    cid = lax.axis_index("c")
    sid = lax.axis_index("s")
    wid = cid * NS + sid
    sems = (sem0, sem1, sem2, sem3)

    pltpu.async_copy(dst_hbm.at[pl.ds(wid * CHUNKS, CHUNKS)], dst_v, sem0)

    def fill(i, carry):
        ones_v[pl.ds(i * 16, 16)] = jnp.ones((16,), jnp.float32)
        return carry

    lax.fori_loop(0, CHUNK // 16, fill, None)

    def fillz(i, carry):
        buf_v[pl.ds(i * 16, 16)] = jnp.zeros((16,), jnp.float32)
        return carry

    lax.fori_loop(0, RPT // 16, fillz, None)
    pltpu.sync_copy(buf_v, hist.at[pl.ds(sid * RPT, RPT)])
    pltpu.make_async_copy(dst_hbm.at[pl.ds(wid * CHUNKS, CHUNKS)], dst_v, sem0).wait()
    plsc.subcore_barrier()

    def fire(j, b):
        pltpu.async_copy(ones_v, hist.at[dst_v.at[j]], sems[b], add=True)

    for j in range(4):
        fire(j, j)

    def body(i, carry):
        for b in range(4):
            j = 4 * i + b
            pltpu.make_async_copy(ones_v, hist.at[dst_v.at[j]], sems[b]).wait()

            @pl.when(j + 4 < CHUNKS)
            def _():
                fire(j + 4, b)

        return carry

    lax.fori_loop(0, CHUNKS // 4, body, None)
    plsc.subcore_barrier()

    sl = pl.ds(sid * RPT, RPT)

    @pl.when(cid == 0)
    def _():
        pltpu.sync_copy(hist.at[sl], out0.at[sl])

    @pl.when(cid == 1)
    def _():
        pltpu.sync_copy(hist.at[sl], out1.at[sl])


_sc_deg = pl.kernel(
    _sc_deg_body,
    out_type=(jax.ShapeDtypeStruct((NPAD,), jnp.float32),
              jax.ShapeDtypeStruct((NPAD,), jnp.float32)),
    mesh=_MESH,
    scratch_types=[
        pltpu.VMEM((CHUNKS, CHUNK), jnp.int32),
        pltpu.VMEM((CHUNK,), jnp.float32),
        pltpu.VMEM((RPT,), jnp.float32),
        pltpu.VMEM_SHARED((NPAD,), jnp.float32),
        pltpu.SemaphoreType.DMA,
        pltpu.SemaphoreType.DMA,
        pltpu.SemaphoreType.DMA,
        pltpu.SemaphoreType.DMA,
    ],
)


def _expand(rows_bf, rows_f):
    himask = jnp.full((D // 2,), -65536, jnp.int32)

    def body(r, carry):
        w = rows_bf[r, pl.ds(0, D // 2)]
        lo = lax.bitcast_convert_type(lax.shift_left(w, 16), jnp.float32)
        hi = lax.bitcast_convert_type(lax.bitwise_and(w, himask), jnp.float32)
        rows_f[r, pl.ds(0, D // 2)] = lo
        rows_f[r, pl.ds(D // 2, D // 2)] = hi
        return carry

    lax.fori_loop(0, CHUNK, body, None)


def _sc_agg_body(yb_hbm, src_hbm, dst_hbm, out0, out1,
                 src_v, dst_v, bf0, bf1, rf0, rf1, partial,
                 gsem0, gsem1, ssem0, ssem1):
    cid = lax.axis_index("c")
    sid = lax.axis_index("s")
    wid = cid * NS + sid
    bf = (bf0, bf1)
    rf = (rf0, rf1)
    gsem = (gsem0, gsem1)
    ssem = (ssem0, ssem1)

    _zero_rows(rf0, CHUNK)
    nz = RPT // 64
    for k in range(nz):
        pltpu.async_copy(
            rf0.at[pl.ds(0, 64)],
            partial.at[pl.ds(sid * RPT + k * 64, 64)], ssem0)
    for k in range(nz):
        pltpu.make_async_copy(
            rf0.at[pl.ds(0, 64)],
            partial.at[pl.ds(sid * RPT + k * 64, 64)], ssem0).wait()
    plsc.subcore_barrier()

    def start_g(b, j):
        pltpu.async_copy(yb_hbm.at[src_v.at[j]], bf[b], gsem[b])

    def wait_g(b, j):
        pltpu.make_async_copy(yb_hbm.at[src_v.at[j]], bf[b], gsem[b]).wait()

    def start_s(b, j):
        pltpu.async_copy(rf[b], partial.at[dst_v.at[j]], ssem[b], add=True)

    def wait_s(b, j):
        pltpu.make_async_copy(rf[b], partial.at[dst_v.at[j]], ssem[b]).wait()

    for h in range(CHUNKS // HALFN):
        base = wid * CHUNKS + h * HALFN
        pltpu.sync_copy(src_hbm.at[pl.ds(base, HALFN)], src_v)
        pltpu.sync_copy(dst_hbm.at[pl.ds(base, HALFN)], dst_v)
        start_g(0, 0)
        start_g(1, 1)

        def body(i, carry):
            for b in range(2):
                j = 2 * i + b
                wait_g(b, j)

                @pl.when(j >= 2)
                def _():
                    wait_s(b, j - 2)

                _expand(bf[b], rf[b])

                @pl.when(j + 2 < HALFN)
                def _():
                    start_g(b, j + 2)

                start_s(b, j)
            return carry

        lax.fori_loop(0, HALFN // 2, body, None)
        wait_s(0, HALFN - 2)
        wait_s(1, HALFN - 1)
    plsc.subcore_barrier()

    sl = pl.ds(sid * RPT, RPT)

    @pl.when(cid == 0)
    def _():
        pltpu.sync_copy(partial.at[sl], out0.at[sl])

    @pl.when(cid == 1)
    def _():
        pltpu.sync_copy(partial.at[sl], out1.at[sl])


_sc_agg = pl.kernel(
    _sc_agg_body,
    out_type=(jax.ShapeDtypeStruct((NPAD, D), jnp.float32),
              jax.ShapeDtypeStruct((NPAD, D), jnp.float32)),
    mesh=_MESH,
    scratch_types=[
        pltpu.VMEM((HALFN, CHUNK), jnp.int32),
        pltpu.VMEM((HALFN, CHUNK), jnp.int32),
        pltpu.VMEM((CHUNK, D // 2), jnp.int32),
        pltpu.VMEM((CHUNK, D // 2), jnp.int32),
        pltpu.VMEM((CHUNK, D), jnp.float32),
        pltpu.VMEM((CHUNK, D), jnp.float32),
        pltpu.VMEM_SHARED((NPAD, D), jnp.float32),
        pltpu.SemaphoreType.DMA,
        pltpu.SemaphoreType.DMA,
        pltpu.SemaphoreType.DMA,
        pltpu.SemaphoreType.DMA,
    ],
    compiler_params=pltpu.CompilerParams(use_tc_tiling_on_sc=False),
)


def _dis(h0, h1):
    return lax.rsqrt(h0[...] + h1[...] + 1.0)


def _tc_first_body(x_ref, w_ref, wp_ref, h0_ref, h1_ref, o_ref, ob_ref):
    dis = _dis(h0_ref, h1_ref)
    xv = x_ref[...]
    o_ref[...] = jnp.dot(xv, w_ref[...],
                         preferred_element_type=jnp.float32) * dis
    ob_ref[...] = (jnp.dot(xv, wp_ref[...],
                           preferred_element_type=jnp.float32)
                   * dis).astype(jnp.bfloat16)


def _tc_mid_body(y_ref, p0_ref, p1_ref, h0_ref, h1_ref, b_ref, w_ref, wp_ref,
                 o_ref, ob_ref):
    dis = _dis(h0_ref, h1_ref)
    h = dis * (p0_ref[...] + p1_ref[...] + y_ref[...]) + b_ref[...]
    h = jnp.maximum(h, 0.0)
    o_ref[...] = jnp.dot(h, w_ref[...],
                         preferred_element_type=jnp.float32) * dis
    ob_ref[...] = (jnp.dot(h, wp_ref[...],
                           preferred_element_type=jnp.float32)
                   * dis).astype(jnp.bfloat16)


def _tc_last_body(y_ref, p0_ref, p1_ref, h0_ref, h1_ref, b_ref, o_ref):
    dis = _dis(h0_ref, h1_ref)
    o_ref[...] = dis * (p0_ref[...] + p1_ref[...] + y_ref[...]) + b_ref[...]


_GRID = N // TCB
_row_spec = pl.BlockSpec((TCB, D), lambda i: (i, 0))
_w_spec = pl.BlockSpec((D, D), lambda i: (0, 0))
_h_spec = pl.BlockSpec((TCB, 1), lambda i: (i, 0))
_b_spec = pl.BlockSpec((1, D), lambda i: (0, 0))
_out_f32 = jax.ShapeDtypeStruct((N, D), jnp.float32)
_out_bf16 = jax.ShapeDtypeStruct((N, D), jnp.bfloat16)

_tc_first = pl.pallas_call(
    _tc_first_body, grid=(_GRID,),
    in_specs=[_row_spec, _w_spec, _w_spec, _h_spec, _h_spec],
    out_specs=(_row_spec, _row_spec), out_shape=(_out_f32, _out_bf16))

_tc_mid = pl.pallas_call(
    _tc_mid_body, grid=(_GRID,),
    in_specs=[_row_spec, _row_spec, _row_spec, _h_spec, _h_spec, _b_spec,
              _w_spec, _w_spec],
    out_specs=(_row_spec, _row_spec), out_shape=(_out_f32, _out_bf16))

_tc_last = pl.pallas_call(
    _tc_last_body, grid=(_GRID,),
    in_specs=[_row_spec, _row_spec, _row_spec, _h_spec, _h_spec, _b_spec],
    out_specs=_row_spec, out_shape=_out_f32)


def kernel(x, W1, b1, W2, b2, W3, b3, edge_index):
    src = edge_index[0].astype(jnp.int32)
    dst = edge_index[1].astype(jnp.int32)
    npad_e = EP - E
    pad = jnp.arange(npad_e, dtype=jnp.int32)
    src_p = jnp.concatenate([src, pad % N]).reshape(EP // CHUNK, CHUNK)
    dst_p = jnp.concatenate([dst, N + pad % (NPAD - N)]).reshape(EP // CHUNK, CHUNK)

    perm = jnp.asarray(_PERM)
    W1p = W1[:, perm]
    W2p = W2[:, perm]
    W3p = W3[:, perm]

    h0, h1 = _sc_deg(dst_p)
    h0 = h0.reshape(NPAD, 1)
    h1 = h1.reshape(NPAD, 1)
    b1r = b1.reshape(1, D)
    b2r = b2.reshape(1, D)
    b3r = b3.reshape(1, D)

    def pack(yb):
        return lax.bitcast_convert_type(yb.reshape(N, D // 2, 2), jnp.int32)

    y1, yb1 = _tc_first(x, W1, W1p, h0, h1)
    p0, p1 = _sc_agg(pack(yb1), src_p, dst_p)
    y2, yb2 = _tc_mid(y1, p0, p1, h0, h1, b1r, W2, W2p)
    p0, p1 = _sc_agg(pack(yb2), src_p, dst_p)
    y3, yb3 = _tc_mid(y2, p0, p1, h0, h1, b2r, W3, W3p)
    p0, p1 = _sc_agg(pack(yb3), src_p, dst_p)
    return _tc_last(y3, p0, p1, h0, h1, b3r)


# speedup vs baseline: 1.0271x; 1.0271x over previous
import functools

import jax
import jax.numpy as jnp
import numpy as np
from jax import lax
from jax.experimental import pallas as pl
from jax.experimental.pallas import tpu as pltpu
from jax.experimental.pallas import tpu_sc as plsc

N = 10000
D = 128
E = 320000
NC = 2
NS = 16
NW = NC * NS
CHUNK = 96
CHUNKS = 112
HALFN = 56
EP = NW * CHUNKS * CHUNK
NPAD = 10240
RPT = NPAD // NS
TCB = 2000

_PERM = np.array([c // 2 if c % 2 == 0 else 64 + c // 2 for c in range(D)],
                 np.int32)

_MESH = plsc.VectorSubcoreMesh(
    core_axis_name="c", subcore_axis_name="s", num_cores=NC, num_subcores=NS)


def _zero_rows(ref, nrows):
    zeros16 = jnp.zeros((16,), jnp.float32)

    def body(i, carry):
        for j in range(8):
            ref[i, pl.ds(j * 16, 16)] = zeros16
        return carry

    lax.fori_loop(0, nrows, body, None)


def _sc_deg_body(dst_hbm, out0, out1, dst_v, ones_v, buf_v, hist,
                 sem0, sem1, sem2, sem3):
    cid = lax.axis_index("c")
    sid = lax.axis_index("s")
    wid = cid * NS + sid
    sems = (sem0, sem1, sem2, sem3)

    pltpu.async_copy(dst_hbm.at[pl.ds(wid * CHUNKS, CHUNKS)], dst_v, sem0)

    def fill(i, carry):
        ones_v[pl.ds(i * 16, 16)] = jnp.ones((16,), jnp.float32)
        return carry

    lax.fori_loop(0, CHUNK // 16, fill, None)

    def fillz(i, carry):
        buf_v[pl.ds(i * 16, 16)] = jnp.zeros((16,), jnp.float32)
        return carry

    lax.fori_loop(0, RPT // 16, fillz, None)
    pltpu.sync_copy(buf_v, hist.at[pl.ds(sid * RPT, RPT)])
    pltpu.make_async_copy(dst_hbm.at[pl.ds(wid * CHUNKS, CHUNKS)], dst_v, sem0).wait()
    plsc.subcore_barrier()

    def fire(j, b):
        pltpu.async_copy(ones_v, hist.at[dst_v.at[j]], sems[b], add=True)

    for j in range(4):
        fire(j, j)

    def body(i, carry):
        for b in range(4):
            j = 4 * i + b
            pltpu.make_async_copy(ones_v, hist.at[dst_v.at[j]], sems[b]).wait()

            @pl.when(j + 4 < CHUNKS)
            def _():
                fire(j + 4, b)

        return carry

    lax.fori_loop(0, CHUNKS // 4, body, None)
    plsc.subcore_barrier()

    sl = pl.ds(sid * RPT, RPT)

    @pl.when(cid == 0)
    def _():
        pltpu.sync_copy(hist.at[sl], out0.at[sl])

    @pl.when(cid == 1)
    def _():
        pltpu.sync_copy(hist.at[sl], out1.at[sl])


_sc_deg = pl.kernel(
    _sc_deg_body,
    out_type=(jax.ShapeDtypeStruct((NPAD,), jnp.float32),
              jax.ShapeDtypeStruct((NPAD,), jnp.float32)),
    mesh=_MESH,
    scratch_types=[
        pltpu.VMEM((CHUNKS, CHUNK), jnp.int32),
        pltpu.VMEM((CHUNK,), jnp.float32),
        pltpu.VMEM((RPT,), jnp.float32),
        pltpu.VMEM_SHARED((NPAD,), jnp.float32),
        pltpu.SemaphoreType.DMA,
        pltpu.SemaphoreType.DMA,
        pltpu.SemaphoreType.DMA,
        pltpu.SemaphoreType.DMA,
    ],
)


def _expand(rows_bf, rows_f):
    himask = jnp.full((D // 2,), -65536, jnp.int32)

    def body(i, carry):
        base = i * 8
        for rr in range(8):
            r = base + rr
            w = rows_bf[r, pl.ds(0, D // 2)]
            lo = lax.bitcast_convert_type(lax.shift_left(w, 16), jnp.float32)
            hi = lax.bitcast_convert_type(lax.bitwise_and(w, himask), jnp.float32)
            rows_f[r, pl.ds(0, D // 2)] = lo
            rows_f[r, pl.ds(D // 2, D // 2)] = hi
        return carry

    lax.fori_loop(0, CHUNK // 8, body, None)


def _sc_agg_body(yb_hbm, src_hbm, dst_hbm, out0, out1,
                 src_v, dst_v, bf0, bf1, rf0, rf1, partial,
                 gsem0, gsem1, ssem0, ssem1):
    cid = lax.axis_index("c")
    sid = lax.axis_index("s")
    wid = cid * NS + sid
    bf = (bf0, bf1)
    rf = (rf0, rf1)
    gsem = (gsem0, gsem1)
    ssem = (ssem0, ssem1)

    _zero_rows(rf0, CHUNK)
    nz = RPT // 64
    for k in range(nz):
        pltpu.async_copy(
            rf0.at[pl.ds(0, 64)],
            partial.at[pl.ds(sid * RPT + k * 64, 64)], ssem0)
    for k in range(nz):
        pltpu.make_async_copy(
            rf0.at[pl.ds(0, 64)],
            partial.at[pl.ds(sid * RPT + k * 64, 64)], ssem0).wait()
    plsc.subcore_barrier()

    def start_g(b, j):
        pltpu.async_copy(yb_hbm.at[src_v.at[j]], bf[b], gsem[b])

    def wait_g(b, j):
        pltpu.make_async_copy(yb_hbm.at[src_v.at[j]], bf[b], gsem[b]).wait()

    def start_s(b, j):
        pltpu.async_copy(rf[b], partial.at[dst_v.at[j]], ssem[b], add=True)

    def wait_s(b, j):
        pltpu.make_async_copy(rf[b], partial.at[dst_v.at[j]], ssem[b]).wait()

    for h in range(CHUNKS // HALFN):
        base = wid * CHUNKS + h * HALFN
        pltpu.sync_copy(src_hbm.at[pl.ds(base, HALFN)], src_v)
        pltpu.sync_copy(dst_hbm.at[pl.ds(base, HALFN)], dst_v)
        start_g(0, 0)
        start_g(1, 1)

        def body(i, carry):
            for b in range(2):
                j = 2 * i + b
                wait_g(b, j)

                @pl.when(j >= 2)
                def _():
                    wait_s(b, j - 2)

                _expand(bf[b], rf[b])

                @pl.when(j + 2 < HALFN)
                def _():
                    start_g(b, j + 2)

                start_s(b, j)
            return carry

        lax.fori_loop(0, HALFN // 2, body, None)
        wait_s(0, HALFN - 2)
        wait_s(1, HALFN - 1)
    plsc.subcore_barrier()

    sl = pl.ds(sid * RPT, RPT)

    @pl.when(cid == 0)
    def _():
        pltpu.sync_copy(partial.at[sl], out0.at[sl])

    @pl.when(cid == 1)
    def _():
        pltpu.sync_copy(partial.at[sl], out1.at[sl])


_sc_agg = pl.kernel(
    _sc_agg_body,
    out_type=(jax.ShapeDtypeStruct((NPAD, D), jnp.float32),
              jax.ShapeDtypeStruct((NPAD, D), jnp.float32)),
    mesh=_MESH,
    scratch_types=[
        pltpu.VMEM((HALFN, CHUNK), jnp.int32),
        pltpu.VMEM((HALFN, CHUNK), jnp.int32),
        pltpu.VMEM((CHUNK, D // 2), jnp.int32),
        pltpu.VMEM((CHUNK, D // 2), jnp.int32),
        pltpu.VMEM((CHUNK, D), jnp.float32),
        pltpu.VMEM((CHUNK, D), jnp.float32),
        pltpu.VMEM_SHARED((NPAD, D), jnp.float32),
        pltpu.SemaphoreType.DMA,
        pltpu.SemaphoreType.DMA,
        pltpu.SemaphoreType.DMA,
        pltpu.SemaphoreType.DMA,
    ],
    compiler_params=pltpu.CompilerParams(use_tc_tiling_on_sc=False),
)


def _dis(h0, h1):
    return lax.rsqrt(h0[...] + h1[...] + 1.0)


def _tc_first_body(x_ref, w_ref, wp_ref, h0_ref, h1_ref, o_ref, ob_ref):
    dis = _dis(h0_ref, h1_ref)
    xv = x_ref[...]
    o_ref[...] = jnp.dot(xv, w_ref[...],
                         preferred_element_type=jnp.float32) * dis
    ob_ref[...] = (jnp.dot(xv, wp_ref[...],
                           preferred_element_type=jnp.float32)
                   * dis).astype(jnp.bfloat16)


def _tc_mid_body(y_ref, p0_ref, p1_ref, h0_ref, h1_ref, b_ref, w_ref, wp_ref,
                 o_ref, ob_ref):
    dis = _dis(h0_ref, h1_ref)
    h = dis * (p0_ref[...] + p1_ref[...] + y_ref[...]) + b_ref[...]
    h = jnp.maximum(h, 0.0)
    o_ref[...] = jnp.dot(h, w_ref[...],
                         preferred_element_type=jnp.float32) * dis
    ob_ref[...] = (jnp.dot(h, wp_ref[...],
                           preferred_element_type=jnp.float32)
                   * dis).astype(jnp.bfloat16)


def _tc_last_body(y_ref, p0_ref, p1_ref, h0_ref, h1_ref, b_ref, o_ref):
    dis = _dis(h0_ref, h1_ref)
    o_ref[...] = dis * (p0_ref[...] + p1_ref[...] + y_ref[...]) + b_ref[...]


_GRID = N // TCB
_row_spec = pl.BlockSpec((TCB, D), lambda i: (i, 0))
_w_spec = pl.BlockSpec((D, D), lambda i: (0, 0))
_h_spec = pl.BlockSpec((TCB, 1), lambda i: (i, 0))
_b_spec = pl.BlockSpec((1, D), lambda i: (0, 0))
_out_f32 = jax.ShapeDtypeStruct((N, D), jnp.float32)
_out_bf16 = jax.ShapeDtypeStruct((N, D), jnp.bfloat16)

_tc_first = pl.pallas_call(
    _tc_first_body, grid=(_GRID,),
    in_specs=[_row_spec, _w_spec, _w_spec, _h_spec, _h_spec],
    out_specs=(_row_spec, _row_spec), out_shape=(_out_f32, _out_bf16))

_tc_mid = pl.pallas_call(
    _tc_mid_body, grid=(_GRID,),
    in_specs=[_row_spec, _row_spec, _row_spec, _h_spec, _h_spec, _b_spec,
              _w_spec, _w_spec],
    out_specs=(_row_spec, _row_spec), out_shape=(_out_f32, _out_bf16))

_tc_last = pl.pallas_call(
    _tc_last_body, grid=(_GRID,),
    in_specs=[_row_spec, _row_spec, _row_spec, _h_spec, _h_spec, _b_spec],
    out_specs=_row_spec, out_shape=_out_f32)


def kernel(x, W1, b1, W2, b2, W3, b3, edge_index):
    src = edge_index[0].astype(jnp.int32)
    dst = edge_index[1].astype(jnp.int32)
    npad_e = EP - E
    pad = jnp.arange(npad_e, dtype=jnp.int32)
    src_p = jnp.concatenate([src, pad % N]).reshape(EP // CHUNK, CHUNK)
    dst_p = jnp.concatenate([dst, N + pad % (NPAD - N)]).reshape(EP // CHUNK, CHUNK)

    perm = jnp.asarray(_PERM)
    W1p = W1[:, perm]
    W2p = W2[:, perm]
    W3p = W3[:, perm]

    h0, h1 = _sc_deg(dst_p)
    h0 = h0.reshape(NPAD, 1)
    h1 = h1.reshape(NPAD, 1)
    b1r = b1.reshape(1, D)
    b2r = b2.reshape(1, D)
    b3r = b3.reshape(1, D)

    def pack(yb):
        return lax.bitcast_convert_type(yb.reshape(N, D // 2, 2), jnp.int32)

    y1, yb1 = _tc_first(x, W1, W1p, h0, h1)
    p0, p1 = _sc_agg(pack(yb1), src_p, dst_p)
    y2, yb2 = _tc_mid(y1, p0, p1, h0, h1, b1r, W2, W2p)
    p0, p1 = _sc_agg(pack(yb2), src_p, dst_p)
    y3, yb3 = _tc_mid(y2, p0, p1, h0, h1, b2r, W3, W3p)
    p0, p1 = _sc_agg(pack(yb3), src_p, dst_p)
    return _tc_last(y3, p0, p1, h0, h1, b3r)


# speedup vs baseline: 1.3405x; 1.3051x over previous
import functools

import jax
import jax.numpy as jnp
from jax import lax
from jax.experimental import pallas as pl
from jax.experimental.pallas import tpu as pltpu
from jax.experimental.pallas import tpu_sc as plsc

N = 10000
D = 128
E = 320000
NC = 2
NS = 16
NW = NC * NS
CHUNK = 128
CHUNKS = 80
HALF = 40
EP = NW * CHUNKS * CHUNK
NPAD = 10240
RPT = NPAD // NS
TCB = 2000

_MESH = plsc.VectorSubcoreMesh(
    core_axis_name="c", subcore_axis_name="s", num_cores=NC, num_subcores=NS)


def _zero_rows(ref, nrows):
    zeros16 = jnp.zeros((16,), jnp.float32)

    def body(i, carry):
        for j in range(8):
            ref[i, pl.ds(j * 16, 16)] = zeros16
        return carry

    lax.fori_loop(0, nrows, body, None)


def _sc_deg_body(dst_hbm, out0, out1, dst_v, ones_v, buf_v, hist,
                 sem0, sem1, sem2, sem3):
    cid = lax.axis_index("c")
    sid = lax.axis_index("s")
    wid = cid * NS + sid
    sems = (sem0, sem1, sem2, sem3)

    pltpu.async_copy(dst_hbm.at[pl.ds(wid * CHUNKS, CHUNKS)], dst_v, sem0)

    def fill(i, carry):
        ones_v[pl.ds(i * 16, 16)] = jnp.ones((16,), jnp.float32)
        buf_v[pl.ds(i * 16, 16)] = jnp.zeros((16,), jnp.float32)
        return carry

    lax.fori_loop(0, CHUNK // 16, fill, None)
    for k in range(RPT // CHUNK):
        pltpu.async_copy(buf_v, hist.at[pl.ds(sid * RPT + k * CHUNK, CHUNK)], sem1)
    for k in range(RPT // CHUNK):
        pltpu.make_async_copy(
            buf_v, hist.at[pl.ds(sid * RPT + k * CHUNK, CHUNK)], sem1).wait()
    pltpu.make_async_copy(dst_hbm.at[pl.ds(wid * CHUNKS, CHUNKS)], dst_v, sem0).wait()
    plsc.subcore_barrier()

    def fire(j, b):
        pltpu.async_copy(ones_v, hist.at[dst_v.at[j]], sems[b], add=True)

    for j in range(4):
        fire(j, j)

    def body(i, carry):
        for b in range(4):
            j = 4 * i + b
            pltpu.make_async_copy(ones_v, hist.at[dst_v.at[j]], sems[b]).wait()

            @pl.when(j + 4 < CHUNKS)
            def _():
                fire(j + 4, b)

        return carry

    lax.fori_loop(0, CHUNKS // 4, body, None)
    plsc.subcore_barrier()

    sl = pl.ds(sid * RPT, RPT)

    @pl.when(cid == 0)
    def _():
        pltpu.sync_copy(hist.at[sl], out0.at[sl])

    @pl.when(cid == 1)
    def _():
        pltpu.sync_copy(hist.at[sl], out1.at[sl])


_sc_deg = pl.kernel(
    _sc_deg_body,
    out_type=(jax.ShapeDtypeStruct((NPAD,), jnp.float32),
              jax.ShapeDtypeStruct((NPAD,), jnp.float32)),
    mesh=_MESH,
    scratch_types=[
        pltpu.VMEM((CHUNKS, CHUNK), jnp.int32),
        pltpu.VMEM((CHUNK,), jnp.float32),
        pltpu.VMEM((CHUNK,), jnp.float32),
        pltpu.VMEM_SHARED((NPAD,), jnp.float32),
        pltpu.SemaphoreType.DMA,
        pltpu.SemaphoreType.DMA,
        pltpu.SemaphoreType.DMA,
        pltpu.SemaphoreType.DMA,
    ],
)


def _sc_agg_body(y_hbm, src_hbm, dst_hbm, out0, out1,
                 src_v, dst_v, rows0, rows1, partial,
                 sem0, sem1, ssem0, ssem1):
    cid = lax.axis_index("c")
    sid = lax.axis_index("s")
    wid = cid * NS + sid

    _zero_rows(rows0, CHUNK)
    for k in range(RPT // CHUNK):
        pltpu.async_copy(rows0, partial.at[pl.ds(sid * RPT + k * CHUNK, CHUNK)], ssem0)
    for k in range(RPT // CHUNK):
        pltpu.make_async_copy(
            rows0, partial.at[pl.ds(sid * RPT + k * CHUNK, CHUNK)], ssem0).wait()
    plsc.subcore_barrier()

    rows = (rows0, rows1)
    gsem = (sem0, sem1)
    ssem = (ssem0, ssem1)

    def wait_g(b, j):
        pltpu.make_async_copy(y_hbm.at[src_v.at[j]], rows[b], gsem[b]).wait()

    def wait_s(b, j):
        pltpu.make_async_copy(rows[b], partial.at[dst_v.at[j]], ssem[b]).wait()

    for h in range(CHUNKS // HALF):
        base = wid * CHUNKS + h * HALF
        pltpu.sync_copy(src_hbm.at[pl.ds(base, HALF)], src_v)
        pltpu.sync_copy(dst_hbm.at[pl.ds(base, HALF)], dst_v)
        pltpu.async_copy(y_hbm.at[src_v.at[0]], rows0, sem0)

        def body(i, carry):
            for b in range(2):
                j = 2 * i + b
                nb = 1 - b

                @pl.when(j >= 1)
                def _():
                    wait_s(nb, j - 1)

                @pl.when(j + 1 < HALF)
                def _():
                    pltpu.async_copy(y_hbm.at[src_v.at[j + 1]], rows[nb], gsem[nb])

                wait_g(b, j)
                pltpu.async_copy(rows[b], partial.at[dst_v.at[j]], ssem[b], add=True)
            return carry

        lax.fori_loop(0, HALF // 2, body, None)
        wait_s(1, HALF - 1)
    plsc.subcore_barrier()

    @pl.when(cid == 0)
    def _():
        for k in range(RPT // CHUNK):
            sl = pl.ds(sid * RPT + k * CHUNK, CHUNK)
            pltpu.async_copy(partial.at[sl], out0.at[sl], sem0)
        for k in range(RPT // CHUNK):
            sl = pl.ds(sid * RPT + k * CHUNK, CHUNK)
            pltpu.make_async_copy(partial.at[sl], out0.at[sl], sem0).wait()

    @pl.when(cid == 1)
    def _():
        for k in range(RPT // CHUNK):
            sl = pl.ds(sid * RPT + k * CHUNK, CHUNK)
            pltpu.async_copy(partial.at[sl], out1.at[sl], sem0)
        for k in range(RPT // CHUNK):
            sl = pl.ds(sid * RPT + k * CHUNK, CHUNK)
            pltpu.make_async_copy(partial.at[sl], out1.at[sl], sem0).wait()


_sc_agg = pl.kernel(
    _sc_agg_body,
    out_type=(jax.ShapeDtypeStruct((NPAD, D), jnp.float32),
              jax.ShapeDtypeStruct((NPAD, D), jnp.float32)),
    mesh=_MESH,
    scratch_types=[
        pltpu.VMEM((HALF, CHUNK), jnp.int32),
        pltpu.VMEM((HALF, CHUNK), jnp.int32),
        pltpu.VMEM((CHUNK, D), jnp.float32),
        pltpu.VMEM((CHUNK, D), jnp.float32),
        pltpu.VMEM_SHARED((NPAD, D), jnp.float32),
        pltpu.SemaphoreType.DMA,
        pltpu.SemaphoreType.DMA,
        pltpu.SemaphoreType.DMA,
        pltpu.SemaphoreType.DMA,
    ],
)


def _dis(h0, h1):
    return lax.rsqrt(h0[...] + h1[...] + 1.0)


def _tc_first_body(x_ref, w_ref, h0_ref, h1_ref, o_ref):
    o_ref[...] = jnp.dot(x_ref[...], w_ref[...],
                         preferred_element_type=jnp.float32) * _dis(h0_ref, h1_ref)


def _tc_mid_body(y_ref, p0_ref, p1_ref, h0_ref, h1_ref, b_ref, w_ref, o_ref):
    dis = _dis(h0_ref, h1_ref)
    h = dis * (p0_ref[...] + p1_ref[...] + y_ref[...]) + b_ref[...]
    h = jnp.maximum(h, 0.0)
    o_ref[...] = jnp.dot(h, w_ref[...],
                         preferred_element_type=jnp.float32) * dis


def _tc_last_body(y_ref, p0_ref, p1_ref, h0_ref, h1_ref, b_ref, o_ref):
    dis = _dis(h0_ref, h1_ref)
    o_ref[...] = dis * (p0_ref[...] + p1_ref[...] + y_ref[...]) + b_ref[...]


_GRID = N // TCB
_row_spec = pl.BlockSpec((TCB, D), lambda i: (i, 0))
_w_spec = pl.BlockSpec((D, D), lambda i: (0, 0))
_h_spec = pl.BlockSpec((TCB, 1), lambda i: (i, 0))
_b_spec = pl.BlockSpec((1, D), lambda i: (0, 0))
_out_shape = jax.ShapeDtypeStruct((N, D), jnp.float32)

_tc_first = pl.pallas_call(
    _tc_first_body, grid=(_GRID,),
    in_specs=[_row_spec, _w_spec, _h_spec, _h_spec],
    out_specs=_row_spec, out_shape=_out_shape)

_tc_mid = pl.pallas_call(
    _tc_mid_body, grid=(_GRID,),
    in_specs=[_row_spec, _row_spec, _row_spec, _h_spec, _h_spec, _b_spec, _w_spec],
    out_specs=_row_spec, out_shape=_out_shape)

_tc_last = pl.pallas_call(
    _tc_last_body, grid=(_GRID,),
    in_specs=[_row_spec, _row_spec, _row_spec, _h_spec, _h_spec, _b_spec],
    out_specs=_row_spec, out_shape=_out_shape)


def kernel(x, W1, b1, W2, b2, W3, b3, edge_index):
    src = edge_index[0].astype(jnp.int32)
    dst = edge_index[1].astype(jnp.int32)
    npad_e = EP - E
    pad = jnp.arange(npad_e, dtype=jnp.int32)
    src_p = jnp.concatenate([src, pad % N]).reshape(EP // CHUNK, CHUNK)
    dst_p = jnp.concatenate([dst, N + pad % (NPAD - N)]).reshape(EP // CHUNK, CHUNK)

    h0, h1 = _sc_deg(dst_p)
    h0 = h0.reshape(NPAD, 1)
    h1 = h1.reshape(NPAD, 1)
    b1r = b1.reshape(1, D)
    b2r = b2.reshape(1, D)
    b3r = b3.reshape(1, D)

    y1 = _tc_first(x, W1, h0, h1)
    p0, p1 = _sc_agg(y1, src_p, dst_p)
    y2 = _tc_mid(y1, p0, p1, h0, h1, b1r, W2)
    p0, p1 = _sc_agg(y2, src_p, dst_p)
    y3 = _tc_mid(y2, p0, p1, h0, h1, b2r, W3)
    p0, p1 = _sc_agg(y3, src_p, dst_p)
    return _tc_last(y3, p0, p1, h0, h1, b3r)


# speedup vs baseline: 1.3429x; 1.0018x over previous
import jax
import jax.numpy as jnp
from jax import lax
from jax.experimental import pallas as pl
from jax.experimental.pallas import tpu as pltpu
from jax.experimental.pallas import tpu_sc as plsc

N = 10000
D = 128
E = 320000
NC = 2
NS = 16
NW = NC * NS
CHUNK = 128
CHUNKS = 80
HALF = 40
EP = NW * CHUNKS * CHUNK
NPAD = 10240
RPT = NPAD // NS
TCB = 2000

_MESH = plsc.VectorSubcoreMesh(
    core_axis_name="c", subcore_axis_name="s", num_cores=NC, num_subcores=NS)


def _zero_rows(ref, nrows):
    zeros16 = jnp.zeros((16,), jnp.float32)

    def body(i, carry):
        for j in range(8):
            ref[i, pl.ds(j * 16, 16)] = zeros16
        return carry

    lax.fori_loop(0, nrows, body, None)


def _sc_deg_body(dst_hbm, out0, out1, dst_v, ones_v, buf_v, hist,
                 sem0, sem1, sem2, sem3):
    cid = lax.axis_index("c")
    sid = lax.axis_index("s")
    wid = cid * NS + sid
    sems = (sem0, sem1, sem2, sem3)

    pltpu.async_copy(dst_hbm.at[pl.ds(wid * CHUNKS, CHUNKS)], dst_v, sem0)

    def fill(i, carry):
        ones_v[pl.ds(i * 16, 16)] = jnp.ones((16,), jnp.float32)
        buf_v[pl.ds(i * 16, 16)] = jnp.zeros((16,), jnp.float32)
        return carry

    lax.fori_loop(0, CHUNK // 16, fill, None)
    for k in range(RPT // CHUNK):
        pltpu.async_copy(buf_v, hist.at[pl.ds(sid * RPT + k * CHUNK, CHUNK)], sem1)
    for k in range(RPT // CHUNK):
        pltpu.make_async_copy(
            buf_v, hist.at[pl.ds(sid * RPT + k * CHUNK, CHUNK)], sem1).wait()
    pltpu.make_async_copy(dst_hbm.at[pl.ds(wid * CHUNKS, CHUNKS)], dst_v, sem0).wait()
    plsc.subcore_barrier()

    def fire(j, b):
        pltpu.async_copy(ones_v, hist.at[dst_v.at[j]], sems[b], add=True)

    for j in range(4):
        fire(j, j)

    def body(i, carry):
        for b in range(4):
            j = 4 * i + b
            pltpu.make_async_copy(ones_v, hist.at[dst_v.at[j]], sems[b]).wait()

            @pl.when(j + 4 < CHUNKS)
            def _():
                fire(j + 4, b)

        return carry

    lax.fori_loop(0, CHUNKS // 4, body, None)
    plsc.subcore_barrier()

    sl = pl.ds(sid * RPT, RPT)

    @pl.when(cid == 0)
    def _():
        pltpu.sync_copy(hist.at[sl], out0.at[sl])

    @pl.when(cid == 1)
    def _():
        pltpu.sync_copy(hist.at[sl], out1.at[sl])


_sc_deg = pl.kernel(
    _sc_deg_body,
    out_type=(jax.ShapeDtypeStruct((NPAD,), jnp.float32),
              jax.ShapeDtypeStruct((NPAD,), jnp.float32)),
    mesh=_MESH,
    scratch_types=[
        pltpu.VMEM((CHUNKS, CHUNK), jnp.int32),
        pltpu.VMEM((CHUNK,), jnp.float32),
        pltpu.VMEM((CHUNK,), jnp.float32),
        pltpu.VMEM_SHARED((NPAD,), jnp.float32),
        pltpu.SemaphoreType.DMA,
        pltpu.SemaphoreType.DMA,
        pltpu.SemaphoreType.DMA,
        pltpu.SemaphoreType.DMA,
    ],
)


def _sc_agg_body(y_hbm, src_hbm, dst_hbm, out0, out1,
                 src_v, dst_v, rows0, rows1, partial,
                 sem0, sem1, ssem0, ssem1):
    cid = lax.axis_index("c")
    sid = lax.axis_index("s")
    wid = cid * NS + sid

    _zero_rows(rows0, CHUNK)
    for k in range(RPT // CHUNK):
        pltpu.async_copy(rows0, partial.at[pl.ds(sid * RPT + k * CHUNK, CHUNK)], ssem0)
    for k in range(RPT // CHUNK):
        pltpu.make_async_copy(
            rows0, partial.at[pl.ds(sid * RPT + k * CHUNK, CHUNK)], ssem0).wait()
    plsc.subcore_barrier()

    rows = (rows0, rows1)
    gsem = (sem0, sem1)
    ssem = (ssem0, ssem1)

    def wait_g(b, j):
        pltpu.make_async_copy(y_hbm.at[src_v.at[j]], rows[b], gsem[b]).wait()

    def wait_s(b, j):
        pltpu.make_async_copy(rows[b], partial.at[dst_v.at[j]], ssem[b]).wait()

    for h in range(CHUNKS // HALF):
        base = wid * CHUNKS + h * HALF
        pltpu.sync_copy(src_hbm.at[pl.ds(base, HALF)], src_v)
        pltpu.sync_copy(dst_hbm.at[pl.ds(base, HALF)], dst_v)
        pltpu.async_copy(y_hbm.at[src_v.at[0]], rows0, sem0)

        def body(i, carry):
            for b in range(2):
                j = 2 * i + b
                nb = 1 - b

                @pl.when(j >= 1)
                def _():
                    wait_s(nb, j - 1)

                @pl.when(j + 1 < HALF)
                def _():
                    pltpu.async_copy(y_hbm.at[src_v.at[j + 1]], rows[nb], gsem[nb])

                wait_g(b, j)
                pltpu.async_copy(rows[b], partial.at[dst_v.at[j]], ssem[b], add=True)
            return carry

        lax.fori_loop(0, HALF // 2, body, None)
        wait_s(1, HALF - 1)
    plsc.subcore_barrier()

    @pl.when(cid == 0)
    def _():
        for k in range(RPT // CHUNK):
            sl = pl.ds(sid * RPT + k * CHUNK, CHUNK)
            pltpu.async_copy(partial.at[sl], out0.at[sl], sem0)
        for k in range(RPT // CHUNK):
            sl = pl.ds(sid * RPT + k * CHUNK, CHUNK)
            pltpu.make_async_copy(partial.at[sl], out0.at[sl], sem0).wait()

    @pl.when(cid == 1)
    def _():
        for k in range(RPT // CHUNK):
            sl = pl.ds(sid * RPT + k * CHUNK, CHUNK)
            pltpu.async_copy(partial.at[sl], out1.at[sl], sem0)
        for k in range(RPT // CHUNK):
            sl = pl.ds(sid * RPT + k * CHUNK, CHUNK)
            pltpu.make_async_copy(partial.at[sl], out1.at[sl], sem0).wait()


_sc_agg = pl.kernel(
    _sc_agg_body,
    out_type=(jax.ShapeDtypeStruct((NPAD, D), jnp.float32),
              jax.ShapeDtypeStruct((NPAD, D), jnp.float32)),
    mesh=_MESH,
    scratch_types=[
        pltpu.VMEM((HALF, CHUNK), jnp.int32),
        pltpu.VMEM((HALF, CHUNK), jnp.int32),
        pltpu.VMEM((CHUNK, D), jnp.float32),
        pltpu.VMEM((CHUNK, D), jnp.float32),
        pltpu.VMEM_SHARED((NPAD, D), jnp.float32),
        pltpu.SemaphoreType.DMA,
        pltpu.SemaphoreType.DMA,
        pltpu.SemaphoreType.DMA,
        pltpu.SemaphoreType.DMA,
    ],
)


def _dis(h0, h1):
    return lax.rsqrt(h0[...] + h1[...] + 1.0)


def _tc_first_body(x_ref, w_ref, h0_ref, h1_ref, o_ref):
    o_ref[...] = jnp.dot(x_ref[...], w_ref[...],
                         preferred_element_type=jnp.float32) * _dis(h0_ref, h1_ref)


def _tc_mid_body(y_ref, p0_ref, p1_ref, h0_ref, h1_ref, b_ref, w_ref, o_ref):
    dis = _dis(h0_ref, h1_ref)
    h = dis * (p0_ref[...] + p1_ref[...] + y_ref[...]) + b_ref[...]
    h = jnp.maximum(h, 0.0)
    o_ref[...] = jnp.dot(h, w_ref[...],
                         preferred_element_type=jnp.float32) * dis


def _tc_last_body(y_ref, p0_ref, p1_ref, h0_ref, h1_ref, b_ref, o_ref):
    dis = _dis(h0_ref, h1_ref)
    o_ref[...] = dis * (p0_ref[...] + p1_ref[...] + y_ref[...]) + b_ref[...]


_GRID = N // TCB
_row_spec = pl.BlockSpec((TCB, D), lambda i: (i, 0))
_w_spec = pl.BlockSpec((D, D), lambda i: (0, 0))
_h_spec = pl.BlockSpec((TCB, 1), lambda i: (i, 0))
_b_spec = pl.BlockSpec((1, D), lambda i: (0, 0))
_out_shape = jax.ShapeDtypeStruct((N, D), jnp.float32)

_tc_first = pl.pallas_call(
    _tc_first_body, grid=(_GRID,),
    in_specs=[_row_spec, _w_spec, _h_spec, _h_spec],
    out_specs=_row_spec, out_shape=_out_shape)

_tc_mid = pl.pallas_call(
    _tc_mid_body, grid=(_GRID,),
    in_specs=[_row_spec, _row_spec, _row_spec, _h_spec, _h_spec, _b_spec, _w_spec],
    out_specs=_row_spec, out_shape=_out_shape)

_tc_last = pl.pallas_call(
    _tc_last_body, grid=(_GRID,),
    in_specs=[_row_spec, _row_spec, _row_spec, _h_spec, _h_spec, _b_spec],
    out_specs=_row_spec, out_shape=_out_shape)


def kernel(x, W1, b1, W2, b2, W3, b3, edge_index):
    src = edge_index[0].astype(jnp.int32)
    dst = edge_index[1].astype(jnp.int32)
    npad_e = EP - E
    pad = jnp.arange(npad_e, dtype=jnp.int32)
    src_p = jnp.concatenate([src, pad % N]).reshape(EP // CHUNK, CHUNK)
    dst_p = jnp.concatenate([dst, N + pad % (NPAD - N)]).reshape(EP // CHUNK, CHUNK)

    h0, h1 = _sc_deg(dst_p)
    h0 = h0.reshape(NPAD, 1)
    h1 = h1.reshape(NPAD, 1)
    b1r = b1.reshape(1, D)
    b2r = b2.reshape(1, D)
    b3r = b3.reshape(1, D)

    y1 = _tc_first(x, W1, h0, h1)
    p0, p1 = _sc_agg(y1, src_p, dst_p)
    y2 = _tc_mid(y1, p0, p1, h0, h1, b1r, W2)
    p0, p1 = _sc_agg(y2, src_p, dst_p)
    y3 = _tc_mid(y2, p0, p1, h0, h1, b2r, W3)
    p0, p1 = _sc_agg(y3, src_p, dst_p)
    return _tc_last(y3, p0, p1, h0, h1, b3r)
